# baseline (speedup 1.0000x reference)
import functools

import jax
import jax.numpy as jnp
from jax import lax
from jax.experimental import pallas as pl
from jax.experimental.pallas import tpu as pltpu
from jax.experimental.pallas import tpu_sc as plsc

D = 64
V = 1000000
CH1 = 512
CHB = 256
B0, B1 = 4096, 200


@jax.jit
def _embedding_lookup(idx_t, table_t, tail2):
    info = plsc.get_sparse_core_info()
    nc = info.num_cores
    nw = nc * info.num_subcores
    v_main = (V // CH1) * CH1
    n_units1 = v_main // CH1
    n_loop1 = n_units1 // nw
    n_units2 = (B0 // CHB) * B1
    n_loop2 = n_units2 // nw

    mesh = plsc.VectorSubcoreMesh(core_axis_name="c", subcore_axis_name="s")
    params = pltpu.CompilerParams(
        use_tc_tiling_on_sc=True, needs_layout_passes=False)

    @functools.partial(
        pl.kernel,
        mesh=mesh,
        out_type=jax.ShapeDtypeStruct((V // 2, 128), jnp.float32),
        scratch_types=[
            pltpu.VMEM((D, CH1), jnp.float32),
            pltpu.VMEM((D, CH1), jnp.float32),
            pltpu.VMEM((CH1 // 2, 128), jnp.float32),
            pltpu.VMEM((32, 128), jnp.float32),
            pltpu.SemaphoreType.DMA,
            pltpu.SemaphoreType.DMA,
        ],
        compiler_params=params,
    )
    def pack(table_hbm, tail_hbm, t256_hbm, in0, in1, obuf, tbuf, s0, s1):
        wid = lax.axis_index("s") * nc + lax.axis_index("c")
        ins = (in0, in1)
        sems = (s0, s1)
        iota = lax.iota(jnp.int32, 16)
        colbase = (iota & 1) * jnp.int32(D)

        def start_in(u, p):
            c0 = u * CH1
            for g in range(8):
                pltpu.make_async_copy(
                    table_hbm.at[pl.ds(8 * g, 8), pl.ds(c0, CH1)],
                    ins[p].at[pl.ds(8 * g, 8)], sems[p]).start()

        def finish(u, p):
            for g in range(8):
                pltpu.make_async_copy(
                    table_hbm.at[pl.ds(8 * g, 8), pl.ds(0, CH1)],
                    ins[p].at[pl.ds(8 * g, 8)], sems[p]).wait()
            src = ins[p]

            def cg_body(cg, c):
                rows = (cg * 16 + iota) >> 1
                for d in range(D):
                    cols = colbase + jnp.int32(d)
                    val = src[d, pl.ds(cg * 16, 16)]
                    plsc.store_scatter(obuf, [rows, cols], val)
                return c

            lax.fori_loop(0, CH1 // 16, cg_body, 0)
            pltpu.sync_copy(obuf, t256_hbm.at[pl.ds(u * (CH1 // 2), CH1 // 2)])

        def unit(j):
            return wid + nw * j

        start_in(unit(0), 0)

        def body(h, c):
            j0 = 2 * h
            start_in(unit(j0 + 1), 1)
            finish(unit(j0), 0)

            @pl.when(h < (n_loop1 - 1) // 2 - 1)
            def _():
                start_in(unit(j0 + 2), 0)

            finish(unit(j0 + 1), 1)
            return c

        lax.fori_loop(0, (n_loop1 - 1) // 2, body, 0)
        start_in(unit(n_loop1 - 1), 0)
        finish(unit(n_loop1 - 1), 0)

        @pl.when(wid == 0)
        def _():
            start_in(n_units1 - 1, 0)
            finish(n_units1 - 1, 0)

        @pl.when(wid == 1)
        def _():
            pltpu.sync_copy(tail_hbm, tbuf)
            pltpu.sync_copy(tbuf, t256_hbm.at[pl.ds(v_main // 2, 32)])

    @functools.partial(
        pl.kernel,
        mesh=mesh,
        out_type=jax.ShapeDtypeStruct((B1, D, B0), jnp.float32),
        scratch_types=[
            pltpu.VMEM((CHB,), jnp.int32),
            pltpu.VMEM((CHB,), jnp.int32),
            pltpu.VMEM((CHB,), jnp.int32),
            pltpu.VMEM((CHB,), jnp.int32),
            pltpu.VMEM((CHB, 128), jnp.float32),
            pltpu.VMEM((CHB, 128), jnp.float32),
            pltpu.VMEM((D, CHB), jnp.float32),
            pltpu.VMEM((D, CHB), jnp.float32),
            pltpu.SemaphoreType.DMA,
            pltpu.SemaphoreType.DMA,
        ],
        compiler_params=params,
    )
    def gather(t256_hbm, idx_hbm, out_hbm,
               ix0, ix1, ie0, ie1, g0, g1, ob0, ob1, s0, s1):
        wid = lax.axis_index("s") * nc + lax.axis_index("c")
        ixs = (ix0, ix1)
        ies = (ie0, ie1)
        gbs = (g0, g1)
        obs = (ob0, ob1)
        sems = (s0, s1)
        iota = lax.iota(jnp.int32, 16)

        def start(u, p):
            t = u // (B0 // CHB)
            bc = u % (B0 // CHB)
            pltpu.sync_copy(idx_hbm.at[pl.ds(t * B0 + bc * CHB, CHB)], ixs[p])

            def shift(rg, c):
                ies[p][pl.ds(rg * 16, 16)] = ixs[p][pl.ds(rg * 16, 16)] >> 1
                return c

            lax.fori_loop(0, CHB // 16, shift, 0)
            pltpu.make_async_copy(t256_hbm.at[ies[p]], gbs[p], sems[p]).start()

        def finish(u, p):
            pltpu.make_async_copy(t256_hbm.at[ies[p]], gbs[p], sems[p]).wait()
            t = u // (B0 // CHB)
            bc = u % (B0 // CHB)
            src = gbs[p]
            ob = obs[p]

            def rg_body(rg, c):
                par64 = (ixs[p][pl.ds(rg * 16, 16)] & 1) * jnp.int32(D)
                rows = rg * 16 + iota
                for d in range(D):
                    val = plsc.load_gather(src, [rows, par64 + jnp.int32(d)])
                    ob[d, pl.ds(rg * 16, 16)] = val
                return c

            lax.fori_loop(0, CHB // 16, rg_body, 0)
            pltpu.sync_copy(ob, out_hbm.at[t, :, pl.ds(bc * CHB, CHB)])

        def unit(j):
            return wid + nw * j

        start(unit(0), 0)

        def body(h, c):
            j0 = 2 * h
            start(unit(j0 + 1), 1)
            finish(unit(j0), 0)

            @pl.when(h < n_loop2 // 2 - 1)
            def _():
                start(unit(j0 + 2), 0)

            finish(unit(j0 + 1), 1)
            return c

        lax.fori_loop(0, n_loop2 // 2, body, 0)

    t256 = pack(table_t, tail2)
    return gather(t256, idx_t)


def kernel(x, table):
    idx_t = x.T.reshape(-1)
    table_t = table.T
    tail2 = table[V - 64:].reshape(32, 128)
    out_t = _embedding_lookup(idx_t, table_t, tail2)
    return out_t.transpose(2, 0, 1)


# speedup vs baseline: 2.1696x; 2.1696x over previous
import functools

import jax
import jax.numpy as jnp
from jax import lax
from jax.experimental import pallas as pl
from jax.experimental.pallas import tpu as pltpu
from jax.experimental.pallas import tpu_sc as plsc

D_MODEL = 64
CHUNK = 512


@jax.jit
def _embedding_lookup(idx, table):
    nw_in, n_chunks, _ = idx.shape
    B = idx.size
    info = plsc.get_sparse_core_info()
    nw = info.num_cores * info.num_subcores
    assert nw_in == nw and n_chunks % 2 == 0
    b_per_w = B // nw
    n_pairs = n_chunks // 2

    mesh = plsc.VectorSubcoreMesh(core_axis_name="c", subcore_axis_name="s")

    @functools.partial(
        pl.kernel,
        mesh=mesh,
        out_type=jax.ShapeDtypeStruct((B, D_MODEL), jnp.float32),
        scratch_types=[
            pltpu.VMEM((n_chunks, CHUNK), jnp.int32),
            pltpu.VMEM((CHUNK, D_MODEL), jnp.float32),
            pltpu.VMEM((CHUNK, D_MODEL), jnp.float32),
            pltpu.SemaphoreType.DMA,
            pltpu.SemaphoreType.DMA,
            pltpu.SemaphoreType.DMA,
            pltpu.SemaphoreType.DMA,
        ],
        compiler_params=pltpu.CompilerParams(use_tc_tiling_on_sc=False),
    )
    def k(table_hbm, idx_hbm, out_hbm, idx_v, rows0, rows1, g0s, g1s, o0s, o1s):
        wid = lax.axis_index("s") * info.num_cores + lax.axis_index("c")
        base = wid * b_per_w

        def gat(g, rows, sem):
            return pltpu.make_async_copy(table_hbm.at[idx_v.at[g]], rows, sem)

        def put(g, rows, sem):
            return pltpu.make_async_copy(
                rows, out_hbm.at[pl.ds(base + g * CHUNK, CHUNK)], sem)

        pltpu.sync_copy(idx_hbm.at[wid], idx_v)
        gat(0, rows0, g0s).start()

        def body(j, carry):
            g0 = 2 * j
            g1 = g0 + 1

            @pl.when(j > 0)
            def _():
                put(g0 - 1, rows1, o1s).wait()

            gat(g1, rows1, g1s).start()
            gat(g0, rows0, g0s).wait()
            put(g0, rows0, o0s).start()

            @pl.when(j < n_pairs - 1)
            def _():
                put(g0, rows0, o0s).wait()
                gat(g0 + 2, rows0, g0s).start()

            gat(g1, rows1, g1s).wait()
            put(g1, rows1, o1s).start()
            return carry

        lax.fori_loop(0, n_pairs, body, 0)
        put(n_chunks - 2, rows0, o0s).wait()
        put(n_chunks - 1, rows1, o1s).wait()

    return k(table, idx)


def kernel(x, table):
    info = plsc.get_sparse_core_info()
    nw = info.num_cores * info.num_subcores
    idx = x.reshape(nw, -1, CHUNK)
    out = _embedding_lookup(idx, table)
    return out.reshape(x.shape + (D_MODEL,))
